# Initial kernel scaffold; baseline (speedup 1.0000x reference)
import jax
import jax.numpy as jnp
from jax.experimental import pallas as pl


def kernel(x, edge_index, edge_attr, W1, b1, W2, b2, W3, b3, Wl, bl):
    raise NotImplementedError("write your pallas kernel here")


# speedup vs baseline: 20.7991x; 20.7991x over previous
import functools

import jax
import jax.numpy as jnp
from jax import lax
from jax.experimental import pallas as pl
from jax.experimental.pallas import tpu as pltpu
from jax.experimental.pallas import tpu_sc as plsc

N_NODES = 10000
N_EDGES = 320000
D_FEAT = 128
HID = 16

NPAD = 10240
NC, NS = 2, 16
NW = NC * NS
EPT = N_EDGES // NW
CH = 80
NCH = EPT // CH
SLICE = NPAD // NS

_MESH = plsc.VectorSubcoreMesh(core_axis_name="c", subcore_axis_name="s")
_SC_PARAMS = pltpu.CompilerParams(needs_layout_passes=False,
                                  use_tc_tiling_on_sc=False)

_F32 = jnp.float32


@functools.partial(
    pl.kernel, mesh=_MESH, compiler_params=_SC_PARAMS,
    out_type=jax.ShapeDtypeStruct((NC, NPAD), _F32),
    scratch_types=[
        pltpu.VMEM_SHARED((NPAD,), _F32),
        pltpu.VMEM((SLICE,), _F32),
        pltpu.VMEM((EPT,), jnp.int32),
        pltpu.VMEM((CH,), jnp.int32),
        pltpu.VMEM((CH,), _F32),
    ],
)
def _deg_kernel(dst_hbm, out_hbm, acc, zbuf, dbuf, idx, ones):
    cid = lax.axis_index("c")
    sid = lax.axis_index("s")
    wid = cid * NS + sid
    for j in range(SLICE // 16):
        zbuf[pl.ds(j * 16, 16)] = jnp.zeros((16,), _F32)
    pltpu.sync_copy(zbuf, acc.at[pl.ds(sid * SLICE, SLICE)])
    for j in range(CH // 16):
        ones[pl.ds(j * 16, 16)] = jnp.full((16,), 1.0, _F32)
    pltpu.sync_copy(dst_hbm.at[pl.ds(wid * EPT, EPT)], dbuf)
    plsc.subcore_barrier()

    def body(j, carry):
        for k in range(CH // 16):
            idx[pl.ds(k * 16, 16)] = dbuf[pl.ds(j * CH + k * 16, 16)]
        pltpu.sync_copy(ones, acc.at[idx], add=True)
        return carry

    lax.fori_loop(0, NCH, body, 0)
    plsc.subcore_barrier()
    pltpu.sync_copy(acc.at[pl.ds(sid * SLICE, SLICE)],
                    out_hbm.at[cid, pl.ds(sid * SLICE, SLICE)])


def _dinv_body(p_ref, o_ref):
    deg = p_ref[0:1, :] + p_ref[1:2, :]
    o_ref[...] = jnp.where(deg > 0, lax.rsqrt(jnp.maximum(deg, 1e-12)), 0.0)


def _dinv_call(degp):
    return pl.pallas_call(
        _dinv_body,
        out_shape=jax.ShapeDtypeStruct((1, NPAD), _F32),
    )(degp)


@functools.partial(
    pl.kernel, mesh=_MESH, compiler_params=_SC_PARAMS,
    out_type=jax.ShapeDtypeStruct((NC, NPAD, HID), _F32),
    scratch_types=[
        pltpu.VMEM_SHARED((NPAD, HID), _F32),
        pltpu.VMEM((SLICE, HID), _F32),
        pltpu.VMEM((EPT,), jnp.int32),
        pltpu.VMEM((EPT,), jnp.int32),
        pltpu.VMEM((CH, HID), _F32),
        pltpu.VMEM((CH,), jnp.int32),
        pltpu.VMEM((CH,), jnp.int32),
        pltpu.SemaphoreType.DMA,
    ],
)
def _agg_kernel(gd_hbm, src_hbm, dst_hbm, out_hbm,
                acc, zbuf, sbuf, dbuf, rows, sidx, didx, sem):
    cid = lax.axis_index("c")
    sid = lax.axis_index("s")
    wid = cid * NS + sid

    def zb(i, carry):
        zbuf[i, :] = jnp.zeros((HID,), _F32)
        return carry

    lax.fori_loop(0, SLICE, zb, 0)
    pltpu.sync_copy(zbuf, acc.at[pl.ds(sid * SLICE, SLICE), :])
    pltpu.sync_copy(src_hbm.at[pl.ds(wid * EPT, EPT)], sbuf)
    pltpu.sync_copy(dst_hbm.at[pl.ds(wid * EPT, EPT)], dbuf)
    plsc.subcore_barrier()

    def body(j, carry):
        for k in range(CH // 16):
            sidx[pl.ds(k * 16, 16)] = sbuf[pl.ds(j * CH + k * 16, 16)]
            didx[pl.ds(k * 16, 16)] = dbuf[pl.ds(j * CH + k * 16, 16)]
        pltpu.async_copy(gd_hbm.at[sidx], rows, sem).wait()
        pltpu.sync_copy(rows, acc.at[didx], add=True)
        return carry

    lax.fori_loop(0, NCH, body, 0)
    plsc.subcore_barrier()
    pltpu.sync_copy(acc.at[pl.ds(sid * SLICE, SLICE), :],
                    out_hbm.at[cid, pl.ds(sid * SLICE, SLICE), :])


def _gd1_body(x_ref, w_ref, dc_ref, o_ref):
    g = jnp.dot(x_ref[...], w_ref[...])
    o_ref[...] = dc_ref[...] * g


def _gd1_call(x, W1, dinvcol):
    return pl.pallas_call(
        _gd1_body,
        out_shape=jax.ShapeDtypeStruct((N_NODES, HID), _F32),
    )(x, W1, dinvcol)


def _layer_body(up_ref, dc_ref, b_ref, w_ref, o_ref):
    u = up_ref[0, :N_NODES, :] + up_ref[1, :N_NODES, :]
    h = dc_ref[...] * u + b_ref[...]
    g = jnp.dot(h, w_ref[...])
    o_ref[...] = dc_ref[...] * g


def _layer_call(up, dinvcol, b, W):
    return pl.pallas_call(
        _layer_body,
        out_shape=jax.ShapeDtypeStruct((N_NODES, HID), _F32),
    )(up, dinvcol, b, W)


def _head_body(up_ref, dc_ref, b_ref, wl_ref, bl_ref, o_ref):
    u = up_ref[0, :N_NODES, :] + up_ref[1, :N_NODES, :]
    h = dc_ref[...] * u + b_ref[...]
    pooled = jnp.mean(h, axis=0, keepdims=True)
    o_ref[...] = jnp.dot(pooled, wl_ref[...]) + bl_ref[...]


def _head_call(up, dinvcol, b3, Wl, blr):
    return pl.pallas_call(
        _head_body,
        out_shape=jax.ShapeDtypeStruct((1, 1), _F32),
    )(up, dinvcol, b3, Wl, blr)


def kernel(x, edge_index, edge_attr, W1, b1, W2, b2, W3, b3, Wl, bl):
    del edge_attr
    esrc = edge_index[0]
    edst = edge_index[1]
    degp = _deg_kernel(edst)
    dinv = _dinv_call(degp)
    dinvcol = dinv[0, :N_NODES].reshape(N_NODES, 1)

    gd = _gd1_call(x, W1, dinvcol)
    up = _agg_kernel(gd, esrc, edst)
    gd = _layer_call(up, dinvcol, b1.reshape(1, HID), W2)
    up = _agg_kernel(gd, esrc, edst)
    gd = _layer_call(up, dinvcol, b2.reshape(1, HID), W3)
    up = _agg_kernel(gd, esrc, edst)
    return _head_call(up, dinvcol, b3.reshape(1, HID), Wl, bl.reshape(1, 1))


# speedup vs baseline: 29.3142x; 1.4094x over previous
import functools

import jax
import jax.numpy as jnp
from jax import lax
from jax.experimental import pallas as pl
from jax.experimental.pallas import tpu as pltpu
from jax.experimental.pallas import tpu_sc as plsc

N_NODES = 10000
N_EDGES = 320000
D_FEAT = 128
HID = 16

NPAD = 10240
NC, NS = 2, 16
NW = NC * NS
EPT = N_EDGES // NW
CH = 80
NCH = EPT // CH
SLICE = NPAD // NS

_MESH = plsc.VectorSubcoreMesh(core_axis_name="c", subcore_axis_name="s")
_SC_PARAMS = pltpu.CompilerParams(needs_layout_passes=False,
                                  use_tc_tiling_on_sc=False)

_F32 = jnp.float32


@functools.partial(
    pl.kernel, mesh=_MESH, compiler_params=_SC_PARAMS,
    out_type=jax.ShapeDtypeStruct((NC, NPAD), _F32),
    scratch_types=[
        pltpu.VMEM_SHARED((NPAD,), _F32),
        pltpu.VMEM((SLICE,), _F32),
        pltpu.VMEM((EPT,), jnp.int32),
        pltpu.VMEM((CH,), jnp.int32),
        pltpu.VMEM((CH,), _F32),
    ],
)
def _deg_kernel(dst_hbm, out_hbm, acc, zbuf, dbuf, idx, ones):
    cid = lax.axis_index("c")
    sid = lax.axis_index("s")
    wid = cid * NS + sid
    for j in range(SLICE // 16):
        zbuf[pl.ds(j * 16, 16)] = jnp.zeros((16,), _F32)
    pltpu.sync_copy(zbuf, acc.at[pl.ds(sid * SLICE, SLICE)])
    for j in range(CH // 16):
        ones[pl.ds(j * 16, 16)] = jnp.full((16,), 1.0, _F32)
    pltpu.sync_copy(dst_hbm.at[pl.ds(wid * EPT, EPT)], dbuf)
    plsc.subcore_barrier()

    def body(j, carry):
        for k in range(CH // 16):
            idx[pl.ds(k * 16, 16)] = dbuf[pl.ds(j * CH + k * 16, 16)]
        pltpu.sync_copy(ones, acc.at[idx], add=True)
        return carry

    lax.fori_loop(0, NCH, body, 0)
    plsc.subcore_barrier()
    pltpu.sync_copy(acc.at[pl.ds(sid * SLICE, SLICE)],
                    out_hbm.at[cid, pl.ds(sid * SLICE, SLICE)])


def _dinv_body(p_ref, o_ref):
    deg = p_ref[0:1, :] + p_ref[1:2, :]
    o_ref[...] = jnp.where(deg > 0, lax.rsqrt(jnp.maximum(deg, 1e-12)), 0.0)


def _dinv_call(degp):
    return pl.pallas_call(
        _dinv_body,
        out_shape=jax.ShapeDtypeStruct((1, NPAD), _F32),
    )(degp)


CH2 = 128
CPT = 80
NEPAD = NW * CPT * CH2
NBUF = 4
NGRP = CPT // NBUF


@functools.partial(
    pl.kernel, mesh=_MESH, compiler_params=_SC_PARAMS,
    out_type=jax.ShapeDtypeStruct((NC, NPAD, HID), _F32),
    scratch_types=[
        pltpu.VMEM_SHARED((NPAD, HID), _F32),
        pltpu.VMEM((SLICE, HID), _F32),
        pltpu.VMEM((CPT, CH2), jnp.int32),
        pltpu.VMEM((CPT, CH2), jnp.int32),
        pltpu.VMEM((NBUF, CH2, HID), _F32),
    ] + [pltpu.SemaphoreType.DMA] * (2 * NBUF),
)
def _agg_kernel(gd_hbm, src_hbm, dst_hbm, out_hbm,
                acc, zbuf, sbuf, dbuf, rows, *sems):
    gsem = sems[:NBUF]
    ssem = sems[NBUF:]
    cid = lax.axis_index("c")
    sid = lax.axis_index("s")
    wid = cid * NS + sid

    def zb(i, carry):
        zbuf[i, :] = jnp.zeros((HID,), _F32)
        return carry

    lax.fori_loop(0, SLICE, zb, 0)
    pltpu.sync_copy(zbuf, acc.at[pl.ds(sid * SLICE, SLICE), :])
    pltpu.sync_copy(src_hbm.at[pl.ds(wid * CPT, CPT), :], sbuf)
    pltpu.sync_copy(dst_hbm.at[pl.ds(wid * CPT, CPT), :], dbuf)
    plsc.subcore_barrier()

    for b in range(NBUF):
        pltpu.async_copy(gd_hbm.at[sbuf.at[b]], rows.at[b], gsem[b])

    def group(g, carry):
        for b in range(NBUF):
            j = g * NBUF + b
            pltpu.make_async_copy(gd_hbm.at[sbuf.at[j]],
                                  rows.at[b], gsem[b]).wait()
            pltpu.async_copy(rows.at[b], acc.at[dbuf.at[j]],
                             ssem[b], add=True)
            pltpu.make_async_copy(rows.at[b], acc.at[dbuf.at[j]],
                                  ssem[b]).wait()
            pltpu.async_copy(gd_hbm.at[sbuf.at[j + NBUF]],
                             rows.at[b], gsem[b])
        return carry

    lax.fori_loop(0, NGRP - 1, group, 0)
    for b in range(NBUF):
        j = (NGRP - 1) * NBUF + b
        pltpu.make_async_copy(gd_hbm.at[sbuf.at[j]], rows.at[b],
                              gsem[b]).wait()
        pltpu.async_copy(rows.at[b], acc.at[dbuf.at[j]], ssem[b], add=True)
        pltpu.make_async_copy(rows.at[b], acc.at[dbuf.at[j]], ssem[b]).wait()

    plsc.subcore_barrier()
    pltpu.sync_copy(acc.at[pl.ds(sid * SLICE, SLICE), :],
                    out_hbm.at[cid, pl.ds(sid * SLICE, SLICE), :])


def _gd1_body(x_ref, w_ref, dc_ref, o_ref):
    g = jnp.dot(x_ref[...], w_ref[...])
    o_ref[...] = dc_ref[...] * g


def _gd1_call(x, W1, dinvcol):
    return pl.pallas_call(
        _gd1_body,
        out_shape=jax.ShapeDtypeStruct((N_NODES, HID), _F32),
    )(x, W1, dinvcol)


def _layer_body(up_ref, dc_ref, b_ref, w_ref, o_ref):
    u = up_ref[0, :N_NODES, :] + up_ref[1, :N_NODES, :]
    h = dc_ref[...] * u + b_ref[...]
    g = jnp.dot(h, w_ref[...])
    o_ref[...] = dc_ref[...] * g


def _layer_call(up, dinvcol, b, W):
    return pl.pallas_call(
        _layer_body,
        out_shape=jax.ShapeDtypeStruct((N_NODES, HID), _F32),
    )(up, dinvcol, b, W)


def _head_body(up_ref, dc_ref, b_ref, wl_ref, bl_ref, o_ref):
    u = up_ref[0, :N_NODES, :] + up_ref[1, :N_NODES, :]
    h = dc_ref[...] * u + b_ref[...]
    pooled = jnp.mean(h, axis=0, keepdims=True)
    o_ref[...] = jnp.dot(pooled, wl_ref[...]) + bl_ref[...]


def _head_call(up, dinvcol, b3, Wl, blr):
    return pl.pallas_call(
        _head_body,
        out_shape=jax.ShapeDtypeStruct((1, 1), _F32),
    )(up, dinvcol, b3, Wl, blr)


def kernel(x, edge_index, edge_attr, W1, b1, W2, b2, W3, b3, Wl, bl):
    del edge_attr
    esrc = edge_index[0]
    edst = edge_index[1]
    npad_e = NEPAD - N_EDGES
    esrc2 = jnp.concatenate(
        [esrc, jnp.zeros((npad_e,), jnp.int32)]).reshape(NW * CPT, CH2)
    edst2 = jnp.concatenate(
        [edst, jnp.full((npad_e,), NPAD - 1, jnp.int32)]).reshape(NW * CPT, CH2)
    degp = _deg_kernel(edst)
    dinv = _dinv_call(degp)
    dinvcol = dinv[0, :N_NODES].reshape(N_NODES, 1)

    gd = _gd1_call(x, W1, dinvcol)
    up = _agg_kernel(gd, esrc2, edst2)
    gd = _layer_call(up, dinvcol, b1.reshape(1, HID), W2)
    up = _agg_kernel(gd, esrc2, edst2)
    gd = _layer_call(up, dinvcol, b2.reshape(1, HID), W3)
    up = _agg_kernel(gd, esrc2, edst2)
    return _head_call(up, dinvcol, b3.reshape(1, HID), Wl, bl.reshape(1, 1))


# speedup vs baseline: 30.3035x; 1.0337x over previous
import functools

import jax
import jax.numpy as jnp
from jax import lax
from jax.experimental import pallas as pl
from jax.experimental.pallas import tpu as pltpu
from jax.experimental.pallas import tpu_sc as plsc

N_NODES = 10000
N_EDGES = 320000
D_FEAT = 128
HID = 16

NPAD = 10240
NC, NS = 2, 16
NW = NC * NS
EPT = N_EDGES // NW
CH = 80
NCH = EPT // CH
SLICE = NPAD // NS

_MESH = plsc.VectorSubcoreMesh(core_axis_name="c", subcore_axis_name="s")
_SC_PARAMS = pltpu.CompilerParams(needs_layout_passes=False,
                                  use_tc_tiling_on_sc=False)

_F32 = jnp.float32


@functools.partial(
    pl.kernel, mesh=_MESH, compiler_params=_SC_PARAMS,
    out_type=jax.ShapeDtypeStruct((NC, NPAD), _F32),
    scratch_types=[
        pltpu.VMEM_SHARED((NPAD,), _F32),
        pltpu.VMEM((SLICE,), _F32),
        pltpu.VMEM((EPT,), jnp.int32),
        pltpu.VMEM((CH,), jnp.int32),
        pltpu.VMEM((CH,), _F32),
    ],
)
def _deg_kernel(dst_hbm, out_hbm, acc, zbuf, dbuf, idx, ones):
    cid = lax.axis_index("c")
    sid = lax.axis_index("s")
    wid = cid * NS + sid
    for j in range(SLICE // 16):
        zbuf[pl.ds(j * 16, 16)] = jnp.zeros((16,), _F32)
    pltpu.sync_copy(zbuf, acc.at[pl.ds(sid * SLICE, SLICE)])
    for j in range(CH // 16):
        ones[pl.ds(j * 16, 16)] = jnp.full((16,), 1.0, _F32)
    pltpu.sync_copy(dst_hbm.at[pl.ds(wid * EPT, EPT)], dbuf)
    plsc.subcore_barrier()

    def body(j, carry):
        for k in range(CH // 16):
            idx[pl.ds(k * 16, 16)] = dbuf[pl.ds(j * CH + k * 16, 16)]
        pltpu.sync_copy(ones, acc.at[idx], add=True)
        return carry

    lax.fori_loop(0, NCH, body, 0)
    plsc.subcore_barrier()
    pltpu.sync_copy(acc.at[pl.ds(sid * SLICE, SLICE)],
                    out_hbm.at[cid, pl.ds(sid * SLICE, SLICE)])


def _dinv_body(p_ref, o_ref):
    deg = p_ref[0:1, :] + p_ref[1:2, :]
    o_ref[...] = jnp.where(deg > 0, lax.rsqrt(jnp.maximum(deg, 1e-12)), 0.0)


def _dinv_call(degp):
    return pl.pallas_call(
        _dinv_body,
        out_shape=jax.ShapeDtypeStruct((1, NPAD), _F32),
    )(degp)


CH2 = 128
CPT = 80
NEPAD = NW * CPT * CH2
NBUF = 4
NGRP = CPT // NBUF


@functools.partial(
    pl.kernel, mesh=_MESH, compiler_params=_SC_PARAMS,
    out_type=jax.ShapeDtypeStruct((NC, NPAD, HID), _F32),
    scratch_types=[
        pltpu.VMEM_SHARED((NPAD, HID), _F32),
        pltpu.VMEM((SLICE, HID), _F32),
        pltpu.VMEM((CPT, CH2), jnp.int32),
        pltpu.VMEM((CPT, CH2), jnp.int32),
        pltpu.VMEM((NBUF, CH2, HID), _F32),
    ] + [pltpu.SemaphoreType.DMA] * (2 * NBUF),
)
def _agg_kernel(gd_hbm, src_hbm, dst_hbm, out_hbm,
                acc, zbuf, sbuf, dbuf, rows, *sems):
    gsem = sems[:NBUF]
    ssem = sems[NBUF:]
    cid = lax.axis_index("c")
    sid = lax.axis_index("s")
    wid = cid * NS + sid

    def zb(i, carry):
        zbuf[i, :] = jnp.zeros((HID,), _F32)
        return carry

    lax.fori_loop(0, SLICE, zb, 0)
    pltpu.sync_copy(zbuf, acc.at[pl.ds(sid * SLICE, SLICE), :])
    pltpu.sync_copy(src_hbm.at[pl.ds(wid * CPT, CPT), :], sbuf)
    pltpu.sync_copy(dst_hbm.at[pl.ds(wid * CPT, CPT), :], dbuf)
    plsc.subcore_barrier()

    for b in range(NBUF):
        pltpu.async_copy(gd_hbm.at[sbuf.at[b]], rows.at[b], gsem[b])

    def group(g, carry):
        for b in range(NBUF):
            j = g * NBUF + b
            pltpu.make_async_copy(gd_hbm.at[sbuf.at[j]],
                                  rows.at[b], gsem[b]).wait()
            pltpu.async_copy(rows.at[b], acc.at[dbuf.at[j]],
                             ssem[b], add=True)
            pltpu.make_async_copy(rows.at[b], acc.at[dbuf.at[j]],
                                  ssem[b]).wait()
            pltpu.async_copy(gd_hbm.at[sbuf.at[j + NBUF]],
                             rows.at[b], gsem[b])
        return carry

    lax.fori_loop(0, NGRP - 1, group, 0)
    for b in range(NBUF):
        j = (NGRP - 1) * NBUF + b
        pltpu.make_async_copy(gd_hbm.at[sbuf.at[j]], rows.at[b],
                              gsem[b]).wait()
        pltpu.async_copy(rows.at[b], acc.at[dbuf.at[j]], ssem[b], add=True)
        pltpu.make_async_copy(rows.at[b], acc.at[dbuf.at[j]], ssem[b]).wait()

    plsc.subcore_barrier()
    pltpu.sync_copy(acc.at[pl.ds(sid * SLICE, SLICE), :],
                    out_hbm.at[cid, pl.ds(sid * SLICE, SLICE), :])


def _gd1_body(x_ref, w_ref, dc_ref, o_ref):
    g = jnp.dot(x_ref[...], w_ref[...])
    o_ref[...] = dc_ref[...] * g


def _gd1_call(x, W1, dinvcol):
    return pl.pallas_call(
        _gd1_body,
        out_shape=jax.ShapeDtypeStruct((N_NODES, HID), _F32),
    )(x, W1, dinvcol)


def _layer_body(up_ref, dc_ref, b_ref, w_ref, o_ref):
    u = up_ref[0, :N_NODES, :] + up_ref[1, :N_NODES, :]
    h = dc_ref[...] * u + b_ref[...]
    g = jnp.dot(h, w_ref[...])
    o_ref[...] = dc_ref[...] * g


def _layer_call(up, dinvcol, b, W):
    return pl.pallas_call(
        _layer_body,
        out_shape=jax.ShapeDtypeStruct((N_NODES, HID), _F32),
    )(up, dinvcol, b, W)


def _head_body(up_ref, dc_ref, b_ref, wl_ref, bl_ref, o_ref):
    u = up_ref[0, :N_NODES, :] + up_ref[1, :N_NODES, :]
    h = dc_ref[...] * u + b_ref[...]
    pooled = jnp.mean(h, axis=0, keepdims=True)
    pb = pooled.astype(jnp.bfloat16).astype(_F32)
    wb = wl_ref[...].astype(jnp.bfloat16).astype(_F32)
    o_ref[...] = jnp.sum(pb * wb, axis=1, keepdims=True) + bl_ref[...]


def _head_call(up, dinvcol, b3, Wlr, blr):
    return pl.pallas_call(
        _head_body,
        out_shape=jax.ShapeDtypeStruct((1, 1), _F32),
    )(up, dinvcol, b3, Wlr, blr)


def kernel(x, edge_index, edge_attr, W1, b1, W2, b2, W3, b3, Wl, bl):
    del edge_attr
    esrc = edge_index[0]
    edst = edge_index[1]
    npad_e = NEPAD - N_EDGES
    esrc2 = jnp.concatenate(
        [esrc, jnp.zeros((npad_e,), jnp.int32)]).reshape(NW * CPT, CH2)
    edst2 = jnp.concatenate(
        [edst, jnp.full((npad_e,), NPAD - 1, jnp.int32)]).reshape(NW * CPT, CH2)
    degp = _deg_kernel(edst)
    dinv = _dinv_call(degp)
    dinvcol = dinv[0, :N_NODES].reshape(N_NODES, 1)

    gd = _gd1_call(x, W1, dinvcol)
    up = _agg_kernel(gd, esrc2, edst2)
    gd = _layer_call(up, dinvcol, b1.reshape(1, HID), W2)
    up = _agg_kernel(gd, esrc2, edst2)
    gd = _layer_call(up, dinvcol, b2.reshape(1, HID), W3)
    up = _agg_kernel(gd, esrc2, edst2)
    return _head_call(up, dinvcol, b3.reshape(1, HID), Wl.reshape(1, HID),
                      bl.reshape(1, 1))
